# Initial kernel scaffold; baseline (speedup 1.0000x reference)
import jax
import jax.numpy as jnp
from jax.experimental import pallas as pl


def kernel(x, levels):
    raise NotImplementedError("write your pallas kernel here")


# speedup vs baseline: 156.9111x; 156.9111x over previous
import functools

import jax
import jax.numpy as jnp
from jax.experimental import pallas as pl

DEPTH = 8
G = 8
GD = 48
D = G * GD
GK = G * DEPTH


def _tpq_kernel(x_ref, wdmat_ref, wdt_ref, v0sum_ref, cvec_ref, arows_ref,
                xq_ref, idx_ref, acc_ref, *, blk):
    x = x_ref[...]
    s = jax.lax.dot_general(
        x, wdmat_ref[...], (((1,), (0,)), ((), ())),
        precision=jax.lax.Precision.HIGHEST,
        preferred_element_type=jnp.float32)
    idx = jnp.zeros((blk, G), jnp.int32)
    bits = []
    for k in range(DEPTH):
        s8 = s[:, 8 * k:8 * k + 8]
        c8 = cvec_ref[:, 8 * k:8 * k + 8]
        bit = (2.0 * s8) > c8
        bits.append(bit.astype(jnp.float32))
        idx = idx + bit.astype(jnp.int32) * (1 << k)
        if k < DEPTH - 1:
            arow = arows_ref[k:k + 1, :]
            s = s - jnp.concatenate([bits[-1]] * DEPTH, axis=1) * arow
    bits64 = jnp.concatenate(bits, axis=1)
    xq = jax.lax.dot_general(
        bits64, wdt_ref[...], (((1,), (0,)), ((), ())),
        precision=jax.lax.Precision.HIGHEST,
        preferred_element_type=jnp.float32) + v0sum_ref[...]
    t = xq - x
    xq_ref[...] = x + t
    idx_ref[...] = idx
    p = jnp.sum(t * t)
    i = pl.program_id(0)

    @pl.when(i == 0)
    def _():
        acc_ref[...] = jnp.full((8, 128), p, jnp.float32)

    @pl.when(i > 0)
    def _():
        acc_ref[...] = acc_ref[...] + p


def kernel(x, levels):
    B, T, _ = x.shape
    x2 = x.reshape(B * T, D)
    n = B * T

    lv = levels.astype(jnp.float32)
    v0 = lv[:, :, 0, :]
    v1 = lv[:, :, 1, :]
    wd = v1 - v0
    eye = jnp.eye(G, dtype=jnp.float32)
    wdmat = jnp.einsum('gkd,gh->gdkh', wd, eye).reshape(D, GK)
    wdt = jnp.einsum('gkd,hg->khgd', wd, eye).reshape(GK, D)
    v0sum = jnp.sum(v0, axis=1).reshape(1, D)
    thr0 = jnp.sum(v1 * v1 - v0 * v0, axis=-1)
    p_jk = jnp.einsum('gjd,gkd->gjk', v0, wd)
    jlt = (jnp.arange(DEPTH)[:, None] < jnp.arange(DEPTH)[None, :])
    c = thr0 + 2.0 * jnp.sum(p_jk * jlt[None], axis=1)
    cvec = c.T.reshape(1, GK)
    a_jk = jnp.einsum('gjd,gkd->gjk', wd, wd)
    arows = jnp.transpose(a_jk, (1, 2, 0)).reshape(DEPTH, GK)

    blk = 512
    grid = n // blk
    xq2, idx2, acc = pl.pallas_call(
        functools.partial(_tpq_kernel, blk=blk),
        grid=(grid,),
        in_specs=[
            pl.BlockSpec((blk, D), lambda i: (i, 0)),
            pl.BlockSpec((D, GK), lambda i: (0, 0)),
            pl.BlockSpec((GK, D), lambda i: (0, 0)),
            pl.BlockSpec((1, D), lambda i: (0, 0)),
            pl.BlockSpec((1, GK), lambda i: (0, 0)),
            pl.BlockSpec((DEPTH, GK), lambda i: (0, 0)),
        ],
        out_specs=[
            pl.BlockSpec((blk, D), lambda i: (i, 0)),
            pl.BlockSpec((blk, G), lambda i: (i, 0)),
            pl.BlockSpec((8, 128), lambda i: (0, 0)),
        ],
        out_shape=[
            jax.ShapeDtypeStruct((n, D), jnp.float32),
            jax.ShapeDtypeStruct((n, G), jnp.int32),
            jax.ShapeDtypeStruct((8, 128), jnp.float32),
        ],
    )(x2, wdmat, wdt, v0sum, cvec, arows)

    total_loss = (2.0 / (B * T * GD)) * acc[0, 0]
    return (xq2.reshape(B, T, D), total_loss, idx2.reshape(B, T, G))
